# baseline (speedup 1.0000x reference)
import functools

import jax
import jax.numpy as jnp
from jax.experimental import pallas as pl
from jax.experimental.pallas import tpu as pltpu


def _convs_kernel(q_ref, w1g_ref, b1_ref, w2g_ref, b2_ref, o_ref,
                  qbig_ref, xeb_ref, xob_ref, *, H, W):
    Wq = 40
    M1 = H * Wq
    H2 = H // 2
    M2 = H2 * Wq
    Ho2 = H2 // 2
    W4 = W // 4

    MB = M1 + 8

    @pl.when(pl.program_id(0) == 0)
    def _():
        qbig_ref[:, 96:128] = jnp.zeros_like(qbig_ref[:, 96:128])
        xeb_ref[...] = jnp.zeros_like(xeb_ref)
        xob_ref[...] = jnp.zeros_like(xob_ref)

    w1g = w1g_ref[...]
    w2g = w2g_ref[...]
    b1v = b1_ref[:, 0:32]
    b2v = b2_ref[:, 0:64]
    dot = functools.partial(jnp.dot, preferred_element_type=jnp.float32)
    for s in range(q_ref.shape[0]):
        qbig_ref[:, 0:32] = q_ref[s, pl.ds(0, MB), :]
        qbig_ref[:, 32:64] = q_ref[s, pl.ds(Wq, MB), :]
        qbig_ref[:, 64:96] = q_ref[s, pl.ds(2 * Wq, MB), :]

        A0 = qbig_ref[pl.ds(0, M1), :]
        A1 = qbig_ref[pl.ds(1, M1), :]
        acc0 = dot(A0, w1g[0])
        acc1 = dot(A0, w1g[1])
        acc2 = dot(A0, w1g[2]) + dot(A1, w1g[3])
        acc3 = dot(A0, w1g[4]) + dot(A1, w1g[5])
        y0 = jnp.maximum(acc0 + b1v, 0.0)
        y1 = jnp.maximum(acc1 + b1v, 0.0)
        y2 = jnp.maximum(acc2 + b1v, 0.0)
        y3 = jnp.maximum(acc3 + b1v, 0.0)
        ev = jnp.maximum(y0, y1)
        od = jnp.maximum(y2, y3)

        for hp in range(H2):
            pe = jnp.maximum(ev[(2 * hp) * Wq:(2 * hp) * Wq + W4, :],
                             ev[(2 * hp + 1) * Wq:(2 * hp + 1) * Wq + W4, :]).astype(jnp.bfloat16)
            po = jnp.maximum(od[(2 * hp) * Wq:(2 * hp) * Wq + W4, :],
                             od[(2 * hp + 1) * Wq:(2 * hp + 1) * Wq + W4, :]).astype(jnp.bfloat16)
            for dh in range(3):
                base = hp + 1 - dh
                if base < 0:
                    continue
                xob_ref[pl.ds(base * Wq + 0, W4), 32 * dh:32 * dh + 32] = pe
                xeb_ref[pl.ds(base * Wq + 1, W4), 32 * dh:32 * dh + 32] = po

        E0 = xeb_ref[pl.ds(0, M2), :]
        E1 = xeb_ref[pl.ds(1, M2), :]
        O0 = xob_ref[pl.ds(0, M2), :]
        O1 = xob_ref[pl.ds(1, M2), :]
        acc_e = dot(E0, w2g[0]) + dot(O0, w2g[1]) + dot(E1, w2g[2])
        acc_o = dot(O0, w2g[0]) + dot(E1, w2g[1]) + dot(O1, w2g[2])
        colmax = jnp.maximum(jnp.maximum(acc_e, acc_o) + b2v, 0.0)
        for hp in range(Ho2):
            r0 = colmax[(2 * hp) * Wq:(2 * hp) * Wq + W4, :]
            r1 = colmax[(2 * hp + 1) * Wq:(2 * hp + 1) * Wq + W4, :]
            o_ref[s, pl.ds(hp * W4, W4), :] = jnp.maximum(r0, r1).astype(o_ref.dtype)


def _pack_w1(w1):
    w = w1[:, :, :32]
    gs = []

    def grp(taps):
        g = jnp.zeros((128, 32), w1.dtype)
        for t, p in taps:
            g = g.at[32 * (t // 3) + 8 * p:32 * (t // 3) + 8 * p + 8, :].set(w[t])
        return g

    for r in range(2):
        gs.append(grp([(3 * dh + dw, r + dw) for dh in range(3) for dw in range(3)]))
    gs.append(grp([(3 * dh + dw, 2 + dw) for dh in range(3) for dw in range(2)]))
    gs.append(grp([(3 * dh + 2, 0) for dh in range(3)]))
    gs.append(grp([(3 * dh + 0, 3) for dh in range(3)]))
    gs.append(grp([(3 * dh + dw, dw - 1) for dh in range(3) for dw in (1, 2)]))
    return jnp.stack(gs)


def _pack_w2(w2):
    w = w2[:, :32, :64]
    gs = []
    for dw in range(3):
        g = jnp.zeros((128, 64), w2.dtype)
        for dh in range(3):
            g = g.at[32 * dh:32 * dh + 32, :].set(w[3 * dh + dw])
        gs.append(g)
    return jnp.stack(gs)


def _fused_convs(q_all, w1, b1, w2, b2, *, H, W):
    N = q_all.shape[0]
    Wq = 40
    P2 = (H // 4) * (W // 4)
    MB = H * Wq + 8
    M2B = (H // 2 + 1) * Wq + 8
    SPS = 4 if N % 4 == 0 else 1
    body = functools.partial(_convs_kernel, H=H, W=W)
    return pl.pallas_call(
        body,
        out_shape=jax.ShapeDtypeStruct((N, P2, 64), jnp.bfloat16),
        grid_spec=pltpu.PrefetchScalarGridSpec(
            num_scalar_prefetch=0,
            grid=(N // SPS,),
            in_specs=[
                pl.BlockSpec((SPS, (H + 3) * Wq, 32), lambda n: (n, 0, 0)),
                pl.BlockSpec((6, 128, 32), lambda n: (0, 0, 0)),
                pl.BlockSpec((1, 128), lambda n: (0, 0)),
                pl.BlockSpec((3, 128, 64), lambda n: (0, 0, 0)),
                pl.BlockSpec((1, 128), lambda n: (0, 0)),
            ],
            out_specs=pl.BlockSpec((SPS, P2, 64), lambda n: (n, 0, 0)),
            scratch_shapes=[
                pltpu.VMEM((MB, 128), jnp.bfloat16),
                pltpu.VMEM((M2B, 128), jnp.bfloat16),
                pltpu.VMEM((M2B, 128), jnp.bfloat16),
            ],
        ),
        compiler_params=pltpu.CompilerParams(
            dimension_semantics=("arbitrary",),
            vmem_limit_bytes=48 * 1024 * 1024,
        ),
    )(q_all, _pack_w1(w1), b1, _pack_w2(w2), b2)


def _tr_kernel(x_ref, emb_ref, o_ref):
    t = jax.lax.dot_general(
        x_ref[0], emb_ref[...],
        dimension_numbers=(((0,), (0,)), ((), ())),
        preferred_element_type=jnp.float32,
    )
    o_ref[0] = t.astype(jnp.bfloat16)


def _transpose_ch(xp3, emb):
    N, Cin, L = xp3.shape
    return pl.pallas_call(
        _tr_kernel,
        out_shape=jax.ShapeDtypeStruct((N, L, 8), jnp.bfloat16),
        grid_spec=pltpu.PrefetchScalarGridSpec(
            num_scalar_prefetch=0,
            grid=(N,),
            in_specs=[
                pl.BlockSpec((1, Cin, L), lambda n: (n, 0, 0)),
                pl.BlockSpec((Cin, 8), lambda n: (0, 0)),
            ],
            out_specs=pl.BlockSpec((1, L, 8), lambda n: (n, 0, 0)),
        ),
        compiler_params=pltpu.CompilerParams(
            dimension_semantics=("arbitrary",),
            vmem_limit_bytes=48 * 1024 * 1024,
        ),
    )(xp3, emb)


def _fc_kernel(x_ref, w1_ref, b1_ref, w2_ref, b2_ref, o_ref, acc_ref):
    k = pl.program_id(0)

    @pl.when(k == 0)
    def _():
        acc_ref[...] = jnp.zeros_like(acc_ref)

    tp = w1_ref.shape[0]
    w = w1_ref[...].reshape(tp * 64, 128)
    acc_ref[...] += jnp.dot(x_ref[...], w, preferred_element_type=jnp.float32)

    @pl.when(k == pl.num_programs(0) - 1)
    def _():
        h = jnp.maximum(acc_ref[...] + b1_ref[...], 0.0)
        z = jnp.sum(h * w2_ref[...], axis=1, keepdims=True) + b2_ref[...]
        o_ref[...] = jax.nn.sigmoid(z)


def _fc_head(x_flat, wfc1, b1, w2_row, b2):
    B, K = x_flat.shape
    P2 = K // 64
    w3 = wfc1.reshape(P2, 128, 128)
    tp = min(64, P2)
    return pl.pallas_call(
        _fc_kernel,
        out_shape=jax.ShapeDtypeStruct((B, 1), jnp.float32),
        grid_spec=pltpu.PrefetchScalarGridSpec(
            num_scalar_prefetch=0,
            grid=(P2 // tp,),
            in_specs=[
                pl.BlockSpec((B, tp * 64), lambda k: (0, k)),
                pl.BlockSpec((tp, 64, 128), lambda k: (k, 0, 0)),
                pl.BlockSpec((1, 128), lambda k: (0, 0)),
                pl.BlockSpec((1, 128), lambda k: (0, 0)),
                pl.BlockSpec((1, 1), lambda k: (0, 0)),
            ],
            out_specs=pl.BlockSpec((B, 1), lambda k: (0, 0)),
            scratch_shapes=[pltpu.VMEM((B, 128), jnp.float32)],
        ),
        compiler_params=pltpu.CompilerParams(
            dimension_semantics=("arbitrary",),
            vmem_limit_bytes=32 * 1024 * 1024,
        ),
    )(x_flat, w3, b1, w2_row, b2)


def kernel(x, w1, b1, w2, b2, wfc1, bfc1, wfc2, bfc2):
    N, Cin, H, W = x.shape
    Wq = 40
    xp = jnp.pad(x, ((0, 0), (0, 0), (1, 2), (1, 4 * Wq - W - 1)))
    emb = jnp.eye(Cin, 8, dtype=x.dtype)
    t = _transpose_ch(xp.reshape(N, Cin, (H + 3) * 4 * Wq), emb)
    q_all = t.reshape(N, (H + 3) * Wq, 32)

    y = _fused_convs(q_all, w1, b1, w2, b2, H=H, W=W)
    x_flat = y.reshape(N, (H // 4) * (W // 4) * 64)
    return _fc_head(x_flat, wfc1, bfc1, wfc2, bfc2)


# speedup vs baseline: 2.2283x; 2.2283x over previous
import functools

import jax
import jax.numpy as jnp
from jax.experimental import pallas as pl
from jax.experimental.pallas import tpu as pltpu


def _convs_kernel(q_ref, w1g_ref, b1_ref, w2g_ref, b2_ref, o_ref,
                  qbig_ref, xeb_ref, xob_ref, *, H, W):
    Wq = 40
    M1 = H * Wq
    H2 = H // 2
    M2 = H2 * Wq
    Ho2 = H2 // 2
    W4 = W // 4

    MB = M1 + 8

    @pl.when(pl.program_id(0) == 0)
    def _():
        qbig_ref[:, 48:128] = jnp.zeros_like(qbig_ref[:, 48:128])
        xeb_ref[...] = jnp.zeros_like(xeb_ref)
        xob_ref[...] = jnp.zeros_like(xob_ref)

    w1g = w1g_ref[...]
    w2g = w2g_ref[...]
    b1v = b1_ref[:, 0:32]
    b2v = b2_ref[:, 0:64]
    dot = functools.partial(jnp.dot, preferred_element_type=jnp.float32)
    for s in range(q_ref.shape[0]):
        qbig_ref[:, 0:16] = q_ref[s, pl.ds(0, MB), :]
        qbig_ref[:, 16:32] = q_ref[s, pl.ds(Wq, MB), :]
        qbig_ref[:, 32:48] = q_ref[s, pl.ds(2 * Wq, MB), :]

        A0 = qbig_ref[pl.ds(0, M1), :]
        A1 = qbig_ref[pl.ds(1, M1), :]
        acc0 = dot(A0, w1g[0])
        acc1 = dot(A0, w1g[1])
        acc2 = dot(A0, w1g[2]) + dot(A1, w1g[3])
        acc3 = dot(A0, w1g[4]) + dot(A1, w1g[5])
        y0 = jnp.maximum(acc0 + b1v, 0.0)
        y1 = jnp.maximum(acc1 + b1v, 0.0)
        y2 = jnp.maximum(acc2 + b1v, 0.0)
        y3 = jnp.maximum(acc3 + b1v, 0.0)
        ev = jnp.maximum(y0, y1)
        od = jnp.maximum(y2, y3)

        for hp in range(H2):
            pe = jnp.maximum(ev[(2 * hp) * Wq:(2 * hp) * Wq + W4, :],
                             ev[(2 * hp + 1) * Wq:(2 * hp + 1) * Wq + W4, :]).astype(jnp.bfloat16)
            po = jnp.maximum(od[(2 * hp) * Wq:(2 * hp) * Wq + W4, :],
                             od[(2 * hp + 1) * Wq:(2 * hp + 1) * Wq + W4, :]).astype(jnp.bfloat16)
            for dh in range(3):
                base = hp + 1 - dh
                if base < 0:
                    continue
                xob_ref[pl.ds(base * Wq + 0, W4), 32 * dh:32 * dh + 32] = pe
                xeb_ref[pl.ds(base * Wq + 1, W4), 32 * dh:32 * dh + 32] = po

        E0 = xeb_ref[pl.ds(0, M2), :]
        E1 = xeb_ref[pl.ds(1, M2), :]
        O0 = xob_ref[pl.ds(0, M2), :]
        O1 = xob_ref[pl.ds(1, M2), :]
        acc_e = dot(E0, w2g[0]) + dot(O0, w2g[1]) + dot(E1, w2g[2])
        acc_o = dot(O0, w2g[0]) + dot(E1, w2g[1]) + dot(O1, w2g[2])
        colmax = jnp.maximum(jnp.maximum(acc_e, acc_o) + b2v, 0.0)
        for hp in range(Ho2):
            r0 = colmax[(2 * hp) * Wq:(2 * hp) * Wq + W4, :]
            r1 = colmax[(2 * hp + 1) * Wq:(2 * hp + 1) * Wq + W4, :]
            o_ref[s, pl.ds(hp * W4, W4), :] = jnp.maximum(r0, r1).astype(o_ref.dtype)


def _pack_w1(w1):
    w = w1[:, :4, :32]
    gs = []

    def grp(taps):
        g = jnp.zeros((128, 32), w1.dtype)
        for t, p in taps:
            g = g.at[16 * (t // 3) + 4 * p:16 * (t // 3) + 4 * p + 4, :].set(w[t])
        return g

    for r in range(2):
        gs.append(grp([(3 * dh + dw, r + dw) for dh in range(3) for dw in range(3)]))
    gs.append(grp([(3 * dh + dw, 2 + dw) for dh in range(3) for dw in range(2)]))
    gs.append(grp([(3 * dh + 2, 0) for dh in range(3)]))
    gs.append(grp([(3 * dh + 0, 3) for dh in range(3)]))
    gs.append(grp([(3 * dh + dw, dw - 1) for dh in range(3) for dw in (1, 2)]))
    return jnp.stack(gs)


def _pack_w2(w2):
    w = w2[:, :32, :64]
    gs = []
    for dw in range(3):
        g = jnp.zeros((128, 64), w2.dtype)
        for dh in range(3):
            g = g.at[32 * dh:32 * dh + 32, :].set(w[3 * dh + dw])
        gs.append(g)
    return jnp.stack(gs)


def _fused_convs(q_all, w1, b1, w2, b2, *, H, W):
    N = q_all.shape[0]
    Wq = 40
    P2 = (H // 4) * (W // 4)
    MB = H * Wq + 8
    M2B = (H // 2 + 1) * Wq + 8
    SPS = 4 if N % 4 == 0 else 1
    body = functools.partial(_convs_kernel, H=H, W=W)
    return pl.pallas_call(
        body,
        out_shape=jax.ShapeDtypeStruct((N, P2, 64), jnp.bfloat16),
        grid_spec=pltpu.PrefetchScalarGridSpec(
            num_scalar_prefetch=0,
            grid=(N // SPS,),
            in_specs=[
                pl.BlockSpec((SPS, (H + 3) * Wq, 16), lambda n: (n, 0, 0)),
                pl.BlockSpec((6, 128, 32), lambda n: (0, 0, 0)),
                pl.BlockSpec((1, 128), lambda n: (0, 0)),
                pl.BlockSpec((3, 128, 64), lambda n: (0, 0, 0)),
                pl.BlockSpec((1, 128), lambda n: (0, 0)),
            ],
            out_specs=pl.BlockSpec((SPS, P2, 64), lambda n: (n, 0, 0)),
            scratch_shapes=[
                pltpu.VMEM((MB, 128), jnp.bfloat16),
                pltpu.VMEM((M2B, 128), jnp.bfloat16),
                pltpu.VMEM((M2B, 128), jnp.bfloat16),
            ],
        ),
        compiler_params=pltpu.CompilerParams(
            dimension_semantics=("arbitrary",),
            vmem_limit_bytes=48 * 1024 * 1024,
        ),
    )(q_all, _pack_w1(w1), b1, _pack_w2(w2), b2)


def _fc_kernel(x_ref, w1_ref, b1_ref, w2_ref, b2_ref, o_ref, acc_ref):
    k = pl.program_id(0)

    @pl.when(k == 0)
    def _():
        acc_ref[...] = jnp.zeros_like(acc_ref)

    tp = w1_ref.shape[0]
    w = w1_ref[...].reshape(tp * 64, 128)
    acc_ref[...] += jnp.dot(x_ref[...], w, preferred_element_type=jnp.float32)

    @pl.when(k == pl.num_programs(0) - 1)
    def _():
        h = jnp.maximum(acc_ref[...] + b1_ref[...], 0.0)
        z = jnp.sum(h * w2_ref[...], axis=1, keepdims=True) + b2_ref[...]
        o_ref[...] = jax.nn.sigmoid(z)


def _fc_head(x_flat, wfc1, b1, w2_row, b2):
    B, K = x_flat.shape
    P2 = K // 64
    w3 = wfc1.reshape(P2, 128, 128)
    tp = min(64, P2)
    return pl.pallas_call(
        _fc_kernel,
        out_shape=jax.ShapeDtypeStruct((B, 1), jnp.float32),
        grid_spec=pltpu.PrefetchScalarGridSpec(
            num_scalar_prefetch=0,
            grid=(P2 // tp,),
            in_specs=[
                pl.BlockSpec((B, tp * 64), lambda k: (0, k)),
                pl.BlockSpec((tp, 64, 128), lambda k: (k, 0, 0)),
                pl.BlockSpec((1, 128), lambda k: (0, 0)),
                pl.BlockSpec((1, 128), lambda k: (0, 0)),
                pl.BlockSpec((1, 1), lambda k: (0, 0)),
            ],
            out_specs=pl.BlockSpec((B, 1), lambda k: (0, 0)),
            scratch_shapes=[pltpu.VMEM((B, 128), jnp.float32)],
        ),
        compiler_params=pltpu.CompilerParams(
            dimension_semantics=("arbitrary",),
            vmem_limit_bytes=32 * 1024 * 1024,
        ),
    )(x_flat, w3, b1, w2_row, b2)


def kernel(x, w1, b1, w2, b2, wfc1, bfc1, wfc2, bfc2):
    N, Cin, H, W = x.shape
    Wq = 40
    xp = jnp.pad(x, ((0, 0), (0, 0), (1, 2), (1, 4 * Wq - W - 1)))
    emb = jnp.eye(Cin, 4, dtype=x.dtype)
    t = jax.lax.dot_general(
        xp.reshape(N, Cin, (H + 3) * 4 * Wq), emb,
        dimension_numbers=(((1,), (0,)), ((), ())),
        preferred_element_type=jnp.float32,
    ).astype(jnp.bfloat16)
    q_all = t.reshape(N, (H + 3) * Wq, 16)

    y = _fused_convs(q_all, w1, b1, w2, b2, H=H, W=W)
    x_flat = y.reshape(N, (H // 4) * (W // 4) * 64)
    return _fc_head(x_flat, wfc1, bfc1, wfc2, bfc2)


# speedup vs baseline: 2.2627x; 1.0155x over previous
import functools

import jax
import jax.numpy as jnp
from jax.experimental import pallas as pl
from jax.experimental.pallas import tpu as pltpu


def _convs_kernel(q_ref, w1g_ref, b1_ref, w2g_ref, b2_ref, o_ref,
                  qbig_ref, xeb_ref, xob_ref, *, H, W):
    Wq = 40
    M1 = H * Wq
    H2 = H // 2
    M2 = H2 * Wq
    Ho2 = H2 // 2
    W4 = W // 4

    MB = M1 + 8

    @pl.when(pl.program_id(0) == 0)
    def _():
        qbig_ref[:, 48:128] = jnp.zeros_like(qbig_ref[:, 48:128])
        xeb_ref[...] = jnp.zeros_like(xeb_ref)
        xob_ref[...] = jnp.zeros_like(xob_ref)

    w1g = w1g_ref[...]
    w2g = w2g_ref[...]
    b1v = b1_ref[:, 0:32]
    b2v = b2_ref[:, 0:64]
    dot = functools.partial(jnp.dot, preferred_element_type=jnp.float32)
    for s in range(q_ref.shape[0]):
        qbig_ref[:, 0:16] = q_ref[s, pl.ds(0, MB), :]
        qbig_ref[:, 16:32] = q_ref[s, pl.ds(Wq, MB), :]
        qbig_ref[:, 32:48] = q_ref[s, pl.ds(2 * Wq, MB), :]

        A0 = qbig_ref[pl.ds(0, M1), :]
        A1 = qbig_ref[pl.ds(1, M1), :]
        acc0 = dot(A0, w1g[0])
        acc1 = dot(A0, w1g[1])
        acc2 = dot(A0, w1g[2]) + dot(A1, w1g[3])
        acc3 = dot(A0, w1g[4]) + dot(A1, w1g[5])
        y0 = jnp.maximum(acc0 + b1v, 0.0)
        y1 = jnp.maximum(acc1 + b1v, 0.0)
        y2 = jnp.maximum(acc2 + b1v, 0.0)
        y3 = jnp.maximum(acc3 + b1v, 0.0)
        ev = jnp.maximum(y0, y1)
        od = jnp.maximum(y2, y3)

        for hp in range(H2):
            pe = jnp.maximum(ev[(2 * hp) * Wq:(2 * hp) * Wq + W4, :],
                             ev[(2 * hp + 1) * Wq:(2 * hp + 1) * Wq + W4, :]).astype(jnp.bfloat16)
            po = jnp.maximum(od[(2 * hp) * Wq:(2 * hp) * Wq + W4, :],
                             od[(2 * hp + 1) * Wq:(2 * hp + 1) * Wq + W4, :]).astype(jnp.bfloat16)
            for dh in range(3):
                base = hp + 1 - dh
                if base < 0:
                    continue
                xob_ref[pl.ds(base * Wq + 0, W4), 32 * dh:32 * dh + 32] = pe
                xeb_ref[pl.ds(base * Wq + 1, W4), 32 * dh:32 * dh + 32] = po

        E0 = xeb_ref[pl.ds(0, M2), :]
        E1 = xeb_ref[pl.ds(1, M2), :]
        O0 = xob_ref[pl.ds(0, M2), :]
        O1 = xob_ref[pl.ds(1, M2), :]
        acc_e = dot(E0, w2g[0]) + dot(O0, w2g[1]) + dot(E1, w2g[2])
        acc_o = dot(O0, w2g[0]) + dot(E1, w2g[1]) + dot(O1, w2g[2])
        colmax = jnp.maximum(jnp.maximum(acc_e, acc_o) + b2v, 0.0)
        for hp in range(Ho2):
            r0 = colmax[(2 * hp) * Wq:(2 * hp) * Wq + W4, :]
            r1 = colmax[(2 * hp + 1) * Wq:(2 * hp + 1) * Wq + W4, :]
            o_ref[s, pl.ds(hp * W4, W4), :] = jnp.maximum(r0, r1).astype(o_ref.dtype)


def _pack_w1(w1):
    w = w1[:, :4, :32]
    gs = []

    def grp(taps):
        g = jnp.zeros((128, 32), w1.dtype)
        for t, p in taps:
            g = g.at[16 * (t // 3) + 4 * p:16 * (t // 3) + 4 * p + 4, :].set(w[t])
        return g

    for r in range(2):
        gs.append(grp([(3 * dh + dw, r + dw) for dh in range(3) for dw in range(3)]))
    gs.append(grp([(3 * dh + dw, 2 + dw) for dh in range(3) for dw in range(2)]))
    gs.append(grp([(3 * dh + 2, 0) for dh in range(3)]))
    gs.append(grp([(3 * dh + 0, 3) for dh in range(3)]))
    gs.append(grp([(3 * dh + dw, dw - 1) for dh in range(3) for dw in (1, 2)]))
    return jnp.stack(gs)


def _pack_w2(w2):
    w = w2[:, :32, :64]
    gs = []
    for dw in range(3):
        g = jnp.zeros((128, 64), w2.dtype)
        for dh in range(3):
            g = g.at[32 * dh:32 * dh + 32, :].set(w[3 * dh + dw])
        gs.append(g)
    return jnp.stack(gs)


def _fused_convs(q_all, w1, b1, w2, b2, *, H, W):
    N = q_all.shape[0]
    Wq = 40
    P2 = (H // 4) * (W // 4)
    MB = H * Wq + 8
    M2B = (H // 2 + 1) * Wq + 8
    SPS = 8 if N % 8 == 0 else 1
    body = functools.partial(_convs_kernel, H=H, W=W)
    return pl.pallas_call(
        body,
        out_shape=jax.ShapeDtypeStruct((N, P2, 64), jnp.bfloat16),
        grid_spec=pltpu.PrefetchScalarGridSpec(
            num_scalar_prefetch=0,
            grid=(N // SPS,),
            in_specs=[
                pl.BlockSpec((SPS, (H + 3) * Wq, 16), lambda n: (n, 0, 0)),
                pl.BlockSpec((6, 128, 32), lambda n: (0, 0, 0)),
                pl.BlockSpec((1, 128), lambda n: (0, 0)),
                pl.BlockSpec((3, 128, 64), lambda n: (0, 0, 0)),
                pl.BlockSpec((1, 128), lambda n: (0, 0)),
            ],
            out_specs=pl.BlockSpec((SPS, P2, 64), lambda n: (n, 0, 0)),
            scratch_shapes=[
                pltpu.VMEM((MB, 128), jnp.bfloat16),
                pltpu.VMEM((M2B, 128), jnp.bfloat16),
                pltpu.VMEM((M2B, 128), jnp.bfloat16),
            ],
        ),
        compiler_params=pltpu.CompilerParams(
            dimension_semantics=("arbitrary",),
            vmem_limit_bytes=48 * 1024 * 1024,
        ),
    )(q_all, _pack_w1(w1), b1, _pack_w2(w2), b2)


def _fc_kernel(x_ref, w1_ref, b1_ref, w2_ref, b2_ref, o_ref, acc_ref):
    k = pl.program_id(0)

    @pl.when(k == 0)
    def _():
        acc_ref[...] = jnp.zeros_like(acc_ref)

    tp = w1_ref.shape[0]
    w = w1_ref[...].reshape(tp * 64, 128)
    acc_ref[...] += jnp.dot(x_ref[...], w, preferred_element_type=jnp.float32)

    @pl.when(k == pl.num_programs(0) - 1)
    def _():
        h = jnp.maximum(acc_ref[...] + b1_ref[...], 0.0)
        z = jnp.sum(h * w2_ref[...], axis=1, keepdims=True) + b2_ref[...]
        o_ref[...] = jax.nn.sigmoid(z)


def _fc_head(x_flat, wfc1, b1, w2_row, b2):
    B, K = x_flat.shape
    P2 = K // 64
    w3 = wfc1.reshape(P2, 128, 128)
    tp = min(64, P2)
    return pl.pallas_call(
        _fc_kernel,
        out_shape=jax.ShapeDtypeStruct((B, 1), jnp.float32),
        grid_spec=pltpu.PrefetchScalarGridSpec(
            num_scalar_prefetch=0,
            grid=(P2 // tp,),
            in_specs=[
                pl.BlockSpec((B, tp * 64), lambda k: (0, k)),
                pl.BlockSpec((tp, 64, 128), lambda k: (k, 0, 0)),
                pl.BlockSpec((1, 128), lambda k: (0, 0)),
                pl.BlockSpec((1, 128), lambda k: (0, 0)),
                pl.BlockSpec((1, 1), lambda k: (0, 0)),
            ],
            out_specs=pl.BlockSpec((B, 1), lambda k: (0, 0)),
            scratch_shapes=[pltpu.VMEM((B, 128), jnp.float32)],
        ),
        compiler_params=pltpu.CompilerParams(
            dimension_semantics=("arbitrary",),
            vmem_limit_bytes=32 * 1024 * 1024,
        ),
    )(x_flat, w3, b1, w2_row, b2)


def kernel(x, w1, b1, w2, b2, wfc1, bfc1, wfc2, bfc2):
    N, Cin, H, W = x.shape
    Wq = 40
    xp = jnp.pad(x, ((0, 0), (0, 0), (1, 2), (1, 4 * Wq - W - 1)))
    emb = jnp.eye(Cin, 4, dtype=x.dtype)
    t = jax.lax.dot_general(
        xp.reshape(N, Cin, (H + 3) * 4 * Wq), emb,
        dimension_numbers=(((1,), (0,)), ((), ())),
        preferred_element_type=jnp.float32,
    ).astype(jnp.bfloat16)
    q_all = t.reshape(N, (H + 3) * Wq, 16)

    y = _fused_convs(q_all, w1, b1, w2, b2, H=H, W=W)
    x_flat = y.reshape(N, (H // 4) * (W // 4) * 64)
    return _fc_head(x_flat, wfc1, bfc1, wfc2, bfc2)
